# baseline (speedup 1.0000x reference)
import functools

import jax
import jax.numpy as jnp
from jax import lax
from jax.experimental import pallas as pl
from jax.experimental.pallas import tpu as pltpu
from jax.experimental.pallas import tpu_sc as plsc

E_EDGES = 160000
C_FEAT = 128
C_PACK = C_FEAT // 2
NB = 4

NUM_CORES = 2
NUM_SUBCORES = 16
NUM_WORKERS = NUM_CORES * NUM_SUBCORES
CHUNK = 80

N_SLICES = 5
E_SLICE = E_EDGES // N_SLICES
E_HALF = E_SLICE // 2

EB = 640
GRID = E_EDGES // EB
SBLK = E_SLICE // EB


def _sc_gather(xpv, idx):
  total = idx.shape[0]
  per_w = total // NUM_WORKERS
  n_chunks = per_w // CHUNK
  assert per_w * NUM_WORKERS == total and n_chunks * CHUNK == per_w
  mesh = plsc.VectorSubcoreMesh(
      core_axis_name="c", subcore_axis_name="s",
      num_cores=NUM_CORES, num_subcores=NUM_SUBCORES)

  @functools.partial(
      pl.kernel,
      mesh=mesh,
      out_type=jax.ShapeDtypeStruct((total, C_PACK), jnp.int32),
      scratch_types=[
          pltpu.VMEM((2, CHUNK), jnp.int32),
          pltpu.VMEM((2, CHUNK, C_PACK), jnp.int32),
          pltpu.SemaphoreType.DMA,
          pltpu.SemaphoreType.DMA,
      ],
      compiler_params=pltpu.CompilerParams(use_tc_tiling_on_sc=False),
  )
  def k(x_hbm, idx_hbm, out_hbm, idx_v, rows_v, gsem0, gsem1):
    wid = lax.axis_index("s") * NUM_CORES + lax.axis_index("c")
    base = pl.multiple_of(wid * per_w, CHUNK)
    gsems = (gsem0, gsem1)
    emax = jnp.full((16,), E_EDGES - 1, jnp.int32)
    ezero = jnp.zeros((16,), jnp.int32)
    c_es = jnp.full((16,), E_SLICE, jnp.int32)
    c_eh = jnp.full((16,), E_HALF, jnp.int32)
    c_m = jnp.full((16,), 2 * E_HALF - 1, jnp.int32)

    def load_idx_and_start(c, b):
      start = pl.multiple_of(base + c * CHUNK, CHUNK)
      pltpu.sync_copy(idx_hbm.at[pl.ds(start, CHUNK)], idx_v.at[b])
      ib = idx_v.at[b]
      for v in range(CHUNK // 16):
        seg = ib[pl.ds(v * 16, 16)]
        kk = jnp.minimum(jnp.maximum(seg, ezero), emax)
        s = lax.div(kk, c_es)
        q = kk - s * c_es
        h = lax.div(q, c_eh)
        ib[pl.ds(v * 16, 16)] = s * c_es + 2 * q - c_m * h
      pltpu.make_async_copy(x_hbm.at[ib], rows_v.at[b], gsems[b]).start()

    def wait_and_writeback(c, b):
      pltpu.make_async_copy(x_hbm.at[idx_v.at[b]], rows_v.at[b],
                            gsems[b]).wait()
      start = pl.multiple_of(base + c * CHUNK, CHUNK)
      pltpu.sync_copy(rows_v.at[b], out_hbm.at[pl.ds(start, CHUNK)])

    for b in (0, 1):
      load_idx_and_start(b, b)

    def body(j, carry):
      for b in (0, 1):
        c = 2 * j + b
        wait_and_writeback(c, b)
        load_idx_and_start(c + 2, b)
      return carry

    lax.fori_loop(0, n_chunks // 2 - 1, body, 0)
    for b in (0, 1):
      wait_and_writeback(n_chunks - 2 + b, b)

  return k(xpv, idx)


def _pack_row(v):
  vb = v.astype(jnp.bfloat16)
  lo = jax.lax.bitcast_convert_type(vb[:, :C_PACK], jnp.uint16)
  hi = jax.lax.bitcast_convert_type(vb[:, C_PACK:], jnp.uint16)
  return (hi.astype(jnp.uint32) << 16) | lo.astype(jnp.uint32)


def _pack_body(xa_ref, xb_ref, o_ref):
  pa = _pack_row(xa_ref[...])
  pb = _pack_row(xb_ref[...])
  o_ref[...] = jax.lax.bitcast_convert_type(
      jnp.concatenate([pa, pb], axis=1), jnp.int32)


def _tc_pack(x):
  nb_half = EB // 2

  def map_a(i):
    return (i + (i // SBLK) * SBLK, 0)

  def map_b(i):
    return (i + (i // SBLK) * SBLK + SBLK, 0)

  return pl.pallas_call(
      _pack_body,
      grid=(GRID,),
      in_specs=[
          pl.BlockSpec((nb_half, C_FEAT), map_a),
          pl.BlockSpec((nb_half, C_FEAT), map_b),
      ],
      out_specs=pl.BlockSpec((nb_half, C_FEAT), lambda i: (i, 0)),
      out_shape=jax.ShapeDtypeStruct((E_EDGES // 2, C_FEAT), jnp.int32),
  )(x, x)


def _split2(d):
  u = jax.lax.bitcast_convert_type(d, jnp.uint32)
  vlo = jax.lax.bitcast_convert_type(
      (u & jnp.uint32(0xFFFF)).astype(jnp.uint16), jnp.bfloat16)
  vhi = jax.lax.bitcast_convert_type(
      (u >> 16).astype(jnp.uint16), jnp.bfloat16)
  a = jnp.concatenate([vlo[:, :C_PACK], vhi[:, :C_PACK]], axis=1)
  b = jnp.concatenate([vlo[:, C_PACK:], vhi[:, C_PACK:]], axis=1)
  return a, b


def _mm_body(xp_ref, d1l_ref, d1h_ref, d2l_ref, d2h_ref, wt_ref, *rest):
  if len(rest) == 5:
    _, _, yl_ref, yh_ref, st_ref = rest
  else:
    yl_ref, yh_ref, st_ref = rest
  i = pl.program_id(0)
  x_lo, x_hi = _split2(xp_ref[...])
  n0_lo, n1_lo = _split2(d1l_ref[...])
  n0_hi, n1_hi = _split2(d1h_ref[...])
  n2_lo, n3_lo = _split2(d2l_ref[...])
  n2_hi, n3_hi = _split2(d2h_ref[...])

  def linear(feats):
    y = jnp.zeros((EB // 2, C_FEAT), jnp.float32)
    for j, f in enumerate(feats):
      y = y + jnp.dot(f, wt_ref[j * C_FEAT:(j + 1) * C_FEAT, :],
                      preferred_element_type=jnp.float32)
    return y

  y_lo = linear((x_lo,
                 jnp.minimum(n0_lo, n1_lo), jnp.maximum(n0_lo, n1_lo),
                 jnp.minimum(n2_lo, n3_lo), jnp.maximum(n2_lo, n3_lo)))
  y_hi = linear((x_hi,
                 jnp.minimum(n0_hi, n1_hi), jnp.maximum(n0_hi, n1_hi),
                 jnp.minimum(n2_hi, n3_hi), jnp.maximum(n2_hi, n3_hi)))
  yl_ref[...] = y_lo.astype(jnp.bfloat16)
  yh_ref[...] = y_hi.astype(jnp.bfloat16)

  @pl.when(i == 0)
  def _():
    st_ref[...] = jnp.zeros_like(st_ref)

  st_ref[0:1, :] += jnp.sum(y_lo, axis=0, keepdims=True) + jnp.sum(
      y_hi, axis=0, keepdims=True)
  st_ref[1:2, :] += jnp.sum(y_lo * y_lo, axis=0, keepdims=True) + jnp.sum(
      y_hi * y_hi, axis=0, keepdims=True)


def _tc_matmul_stats_slice(xp, g2, wt, y_prev, s):
  nb_half = EB // 2
  assert E_SLICE % nb_half == 0
  d_blk = E_SLICE // nb_half

  in_specs = [
      pl.BlockSpec((nb_half, C_FEAT), lambda i: (s * SBLK + i, 0)),
      pl.BlockSpec((nb_half, C_FEAT), lambda i: (i, 0)),
      pl.BlockSpec((nb_half, C_FEAT), lambda i: (SBLK + i, 0)),
      pl.BlockSpec((nb_half, C_FEAT), lambda i: (d_blk + i, 0)),
      pl.BlockSpec((nb_half, C_FEAT), lambda i: (d_blk + SBLK + i, 0)),
      pl.BlockSpec((5 * C_FEAT, C_FEAT), lambda i: (0, 0)),
  ]
  args = [xp, g2, g2, g2, g2, wt]
  kwargs = {}
  if y_prev is not None:
    small = pl.BlockSpec((8, C_FEAT), lambda i: (0, 0))
    in_specs += [small, small]
    args += list(y_prev)
    kwargs["input_output_aliases"] = {6: 0, 7: 1}
  return pl.pallas_call(
      _mm_body,
      grid=(SBLK,),
      in_specs=in_specs,
      out_specs=[
          pl.BlockSpec((nb_half, C_FEAT), lambda i: (s * SBLK + i, 0)),
          pl.BlockSpec((nb_half, C_FEAT), lambda i: (s * SBLK + i, 0)),
          pl.BlockSpec((8, C_FEAT), lambda i: (0, 0)),
      ],
      out_shape=[
          jax.ShapeDtypeStruct((E_EDGES // 2, C_FEAT), jnp.bfloat16),
          jax.ShapeDtypeStruct((E_EDGES // 2, C_FEAT), jnp.bfloat16),
          jax.ShapeDtypeStruct((8, C_FEAT), jnp.float32),
      ],
      **kwargs,
  )(*args)


def _bn_body(y_ref, st0, st1, st2, st3, st4, gb_ref, *rest):
  if len(rest) == 2:
    _, o_ref = rest
  else:
    (o_ref,) = rest
  st = st0[...] + st1[...] + st2[...] + st3[...] + st4[...]
  inv_e = jnp.float32(1.0 / E_EDGES)
  mean = st[0, :] * inv_e
  var = st[1, :] * inv_e - mean * mean
  inv = lax.rsqrt(var + 1e-5)
  scale = gb_ref[0, :] * inv
  shift = gb_ref[1, :] - mean * scale
  yv = y_ref[...].astype(jnp.float32)
  o_ref[...] = jnp.maximum(yv * scale[None, :] + shift[None, :], 0.0)


def _tc_bn_relu_half(y_half, sts, gb, out_prev, hi):
  hblk = E_HALF // EB
  grid = (E_EDGES // 2) // EB

  def out_map(i):
    return ((i // hblk) * 2 * hblk + hi * hblk + (i % hblk), 0)

  small = pl.BlockSpec((8, C_FEAT), lambda i: (0, 0))
  in_specs = ([pl.BlockSpec((EB, C_FEAT), lambda i: (i, 0))]
              + [small] * (len(sts) + 1))
  args = [y_half] + list(sts) + [gb]
  kwargs = {}
  if out_prev is not None:
    in_specs.append(small)
    args.append(out_prev)
    kwargs["input_output_aliases"] = {len(args) - 1: 0}
  return pl.pallas_call(
      _bn_body,
      grid=(grid,),
      in_specs=in_specs,
      out_specs=pl.BlockSpec((EB, C_FEAT), out_map),
      out_shape=jax.ShapeDtypeStruct((E_EDGES, C_FEAT), jnp.float32),
      **kwargs,
  )(*args)


def kernel(x, nb, W, gamma, beta):
  nbs = nb.astype(jnp.int32).reshape(N_SLICES, E_SLICE, NB)
  idx = jnp.concatenate(
      [nbs[:, :, 0:2].reshape(N_SLICES, 2 * E_SLICE),
       nbs[:, :, 2:4].reshape(N_SLICES, 2 * E_SLICE)], axis=1)
  wt = W.T.astype(jnp.bfloat16)
  xp = _tc_pack(x)
  xpv = xp.reshape(E_EDGES, C_PACK)
  y = None
  sts = []
  for s in range(N_SLICES):
    g = _sc_gather(xpv, idx[s])
    g2 = g.reshape(2 * E_SLICE, C_FEAT)
    y_lo, y_hi, st = _tc_matmul_stats_slice(xp, g2, wt, y, s)
    y = (y_lo, y_hi)
    sts.append(st)
  gb = jnp.zeros((8, C_FEAT), jnp.float32).at[0].set(gamma).at[1].set(beta)
  out = _tc_bn_relu_half(y[0], sts, gb, None, 0)
  return _tc_bn_relu_half(y[1], sts, gb, out, 1)


# speedup vs baseline: 1.1252x; 1.1252x over previous
import functools

import jax
import jax.numpy as jnp
from jax import lax
from jax.experimental import pallas as pl
from jax.experimental.pallas import tpu as pltpu
from jax.experimental.pallas import tpu_sc as plsc

E_EDGES = 160000
C_FEAT = 128
C_PACK = C_FEAT // 2
NB = 4

NUM_CORES = 2
NUM_SUBCORES = 16
NUM_WORKERS = NUM_CORES * NUM_SUBCORES
DCHUNK = 40
PCHUNK = 40

N_SLICES = 5
E_SLICE = E_EDGES // N_SLICES

EB = 640
GRID = E_EDGES // EB
SBLK = E_SLICE // EB

_SC_PARAMS = pltpu.CompilerParams(use_tc_tiling_on_sc=False)
_MESH = dict(core_axis_name="c", subcore_axis_name="s",
             num_cores=NUM_CORES, num_subcores=NUM_SUBCORES)


def _to_bf16_bits(u):
  one = jnp.full((16,), 1, jnp.uint32)
  rnd = jnp.full((16,), 0x7FFF, jnp.uint32)
  s16 = jnp.full((16,), 16, jnp.uint32)
  return (u + rnd + ((u >> s16) & one)) >> s16


def _sc_pack(x):
  n_rows = x.shape[0]
  per_w = n_rows // NUM_WORKERS
  n_chunks = per_w // PCHUNK
  assert per_w * NUM_WORKERS == n_rows and n_chunks * PCHUNK == per_w
  mesh = plsc.VectorSubcoreMesh(**_MESH)

  @functools.partial(
      pl.kernel,
      mesh=mesh,
      out_type=jax.ShapeDtypeStruct((n_rows, C_PACK), jnp.int32),
      scratch_types=[
          pltpu.VMEM((2, PCHUNK, C_FEAT), jnp.float32),
          pltpu.VMEM((PCHUNK, C_PACK), jnp.int32),
          pltpu.SemaphoreType.DMA,
          pltpu.SemaphoreType.DMA,
      ],
      compiler_params=_SC_PARAMS,
  )
  def k(x_hbm, out_hbm, xin, xout, sem0, sem1):
    wid = lax.axis_index("s") * NUM_CORES + lax.axis_index("c")
    base = pl.multiple_of(wid * per_w, PCHUNK)
    sems = (sem0, sem1)
    s16c = jnp.full((16,), 16, jnp.uint32)

    def start_load(c, b):
      start = pl.multiple_of(base + c * PCHUNK, PCHUNK)
      pltpu.make_async_copy(x_hbm.at[pl.ds(start, PCHUNK)], xin.at[b],
                            sems[b]).start()

    def wait_load(b):
      pltpu.make_async_copy(x_hbm.at[pl.ds(0, PCHUNK)], xin.at[b],
                            sems[b]).wait()

    def pack_chunk(c, b):
      def row(r, carry):
        for g in range(C_PACK // 16):
          a = xin[b, r, pl.ds(16 * g, 16)]
          cc = xin[b, r, pl.ds(C_PACK + 16 * g, 16)]
          lo = _to_bf16_bits(jax.lax.bitcast_convert_type(a, jnp.uint32))
          hi = _to_bf16_bits(jax.lax.bitcast_convert_type(cc, jnp.uint32))
          xout[r, pl.ds(16 * g, 16)] = jax.lax.bitcast_convert_type(
              (hi << s16c) | lo, jnp.int32)
        return carry

      lax.fori_loop(0, PCHUNK, row, 0)
      start = pl.multiple_of(base + c * PCHUNK, PCHUNK)
      pltpu.sync_copy(xout, out_hbm.at[pl.ds(start, PCHUNK)])

    for b in (0, 1):
      start_load(b, b)

    def body(j, carry):
      for b in (0, 1):
        c = 2 * j + b
        wait_load(b)
        pack_chunk(c, b)
        start_load(c + 2, b)
      return carry

    lax.fori_loop(0, n_chunks // 2 - 1, body, 0)
    for b in (0, 1):
      wait_load(b)
      pack_chunk(n_chunks - 2 + b, b)

  return k(x)


def _sc_gather(xp, idx):
  total = idx.shape[0]
  n_out = total // 2
  per_w = n_out // NUM_WORKERS
  n_chunks = per_w // DCHUNK
  assert per_w * NUM_WORKERS == n_out and n_chunks * DCHUNK == per_w
  mesh = plsc.VectorSubcoreMesh(**_MESH)

  @functools.partial(
      pl.kernel,
      mesh=mesh,
      out_type=jax.ShapeDtypeStruct((n_out, C_FEAT), jnp.int32),
      scratch_types=[
          pltpu.VMEM((2, 2 * DCHUNK), jnp.int32),
          pltpu.VMEM((2, DCHUNK, C_PACK), jnp.int32),
          pltpu.VMEM((2, DCHUNK, C_PACK), jnp.int32),
          pltpu.SemaphoreType.DMA,
          pltpu.SemaphoreType.DMA,
      ],
      compiler_params=_SC_PARAMS,
  )
  def k(x_hbm, idx_hbm, out_hbm, idx_v, rows_l, rows_r, gsem0, gsem1):
    wid = lax.axis_index("s") * NUM_CORES + lax.axis_index("c")
    base = pl.multiple_of(wid * per_w, DCHUNK)
    gsems = (gsem0, gsem1)
    emax = jnp.full((16,), E_EDGES - 1, jnp.int32)
    ezero = jnp.zeros((16,), jnp.int32)

    def halves(b):
      left = idx_v.at[b, pl.ds(0, DCHUNK)]
      right = idx_v.at[b, pl.ds(DCHUNK, DCHUNK)]
      return ((left, rows_l.at[b]), (right, rows_r.at[b]))

    def load_idx_and_start(c, b):
      start = pl.multiple_of(2 * base + c * 2 * DCHUNK, 2 * DCHUNK)
      pltpu.sync_copy(idx_hbm.at[pl.ds(start, 2 * DCHUNK)], idx_v.at[b])
      ib = idx_v.at[b]
      for v in range(2 * DCHUNK // 16):
        seg = ib[pl.ds(v * 16, 16)]
        ib[pl.ds(v * 16, 16)] = jnp.minimum(jnp.maximum(seg, ezero), emax)
      for iv, dv in halves(b):
        pltpu.make_async_copy(x_hbm.at[iv], dv, gsems[b]).start()

    def wait_and_writeback(c, b):
      for iv, dv in halves(b):
        pltpu.make_async_copy(x_hbm.at[iv], dv, gsems[b]).wait()
      start = pl.multiple_of(base + c * DCHUNK, DCHUNK)
      pltpu.sync_copy(rows_l.at[b],
                      out_hbm.at[pl.ds(start, DCHUNK), pl.ds(0, C_PACK)])
      pltpu.sync_copy(rows_r.at[b],
                      out_hbm.at[pl.ds(start, DCHUNK), pl.ds(C_PACK, C_PACK)])

    for b in (0, 1):
      load_idx_and_start(b, b)

    def body(j, carry):
      for b in (0, 1):
        c = 2 * j + b
        wait_and_writeback(c, b)
        load_idx_and_start(c + 2, b)
      return carry

    lax.fori_loop(0, n_chunks // 2 - 1, body, 0)
    for b in (0, 1):
      wait_and_writeback(n_chunks - 2 + b, b)

  return k(xp, idx)


def _split2(d):
  u = jax.lax.bitcast_convert_type(d, jnp.uint32)
  vlo = jax.lax.bitcast_convert_type(
      (u & jnp.uint32(0xFFFF)).astype(jnp.uint16), jnp.bfloat16)
  vhi = jax.lax.bitcast_convert_type(
      (u >> 16).astype(jnp.uint16), jnp.bfloat16)
  a = jnp.concatenate([vlo[:, :C_PACK], vhi[:, :C_PACK]], axis=1)
  b = jnp.concatenate([vlo[:, C_PACK:], vhi[:, C_PACK:]], axis=1)
  return a, b


def _mm_body(x_ref, d1_ref, d2_ref, wt_ref, *rest):
  if len(rest) == 3:
    _, y_ref, st_ref = rest
  else:
    y_ref, st_ref = rest
  i = pl.program_id(0)
  n0, n1 = _split2(d1_ref[...])
  n2, n3 = _split2(d2_ref[...])
  feats = (x_ref[...].astype(jnp.bfloat16),
           jnp.minimum(n0, n1), jnp.maximum(n0, n1),
           jnp.minimum(n2, n3), jnp.maximum(n2, n3))
  y = jnp.zeros((EB, C_FEAT), jnp.float32)
  for j, f in enumerate(feats):
    y = y + jnp.dot(f, wt_ref[j * C_FEAT:(j + 1) * C_FEAT, :],
                    preferred_element_type=jnp.float32)
  y_ref[...] = y.astype(jnp.bfloat16)

  @pl.when(i == 0)
  def _():
    st_ref[...] = jnp.zeros_like(st_ref)

  st_ref[0:1, :] += jnp.sum(y, axis=0, keepdims=True)
  st_ref[1:2, :] += jnp.sum(y * y, axis=0, keepdims=True)


def _tc_matmul_stats_slice(x, g, wt, y_prev, s):
  in_specs = [
      pl.BlockSpec((EB, C_FEAT), lambda i: (s * SBLK + i, 0)),
      pl.BlockSpec((EB, C_FEAT), lambda i: (i, 0)),
      pl.BlockSpec((EB, C_FEAT), lambda i: (SBLK + i, 0)),
      pl.BlockSpec((5 * C_FEAT, C_FEAT), lambda i: (0, 0)),
  ]
  args = [x, g, g, wt]
  kwargs = {}
  if y_prev is not None:
    in_specs.append(pl.BlockSpec((8, C_FEAT), lambda i: (0, 0)))
    args.append(y_prev)
    kwargs["input_output_aliases"] = {4: 0}
  return pl.pallas_call(
      _mm_body,
      grid=(SBLK,),
      in_specs=in_specs,
      out_specs=[
          pl.BlockSpec((EB, C_FEAT), lambda i: (s * SBLK + i, 0)),
          pl.BlockSpec((8, C_FEAT), lambda i: (0, 0)),
      ],
      out_shape=[
          jax.ShapeDtypeStruct((E_EDGES, C_FEAT), jnp.bfloat16),
          jax.ShapeDtypeStruct((8, C_FEAT), jnp.float32),
      ],
      **kwargs,
  )(*args)


def _bn_body(y_ref, st0, st1, st2, st3, st4, gb_ref, o_ref):
  st = st0[...] + st1[...] + st2[...] + st3[...] + st4[...]
  inv_e = jnp.float32(1.0 / E_EDGES)
  mean = st[0, :] * inv_e
  var = st[1, :] * inv_e - mean * mean
  inv = lax.rsqrt(var + 1e-5)
  scale = gb_ref[0, :] * inv
  shift = gb_ref[1, :] - mean * scale
  yv = y_ref[...].astype(jnp.float32)
  o_ref[...] = jnp.maximum(yv * scale[None, :] + shift[None, :], 0.0)


def _tc_bn_relu(y, sts, gb):
  small = pl.BlockSpec((8, C_FEAT), lambda i: (0, 0))
  return pl.pallas_call(
      _bn_body,
      grid=(GRID,),
      in_specs=[pl.BlockSpec((EB, C_FEAT), lambda i: (i, 0))]
      + [small] * (len(sts) + 1),
      out_specs=pl.BlockSpec((EB, C_FEAT), lambda i: (i, 0)),
      out_shape=jax.ShapeDtypeStruct((E_EDGES, C_FEAT), jnp.float32),
  )(y, *sts, gb)


def kernel(x, nb, W, gamma, beta):
  assert E_SLICE % DCHUNK == 0
  nch = E_SLICE // DCHUNK
  nbs = nb.astype(jnp.int32).reshape(N_SLICES, nch, DCHUNK, NB)
  idx = jnp.concatenate(
      [nbs[..., 0:2].transpose(0, 1, 3, 2).reshape(N_SLICES, 2 * E_SLICE),
       nbs[..., 2:4].transpose(0, 1, 3, 2).reshape(N_SLICES, 2 * E_SLICE)],
      axis=1)
  wt = W.T.astype(jnp.bfloat16)
  xp = _sc_pack(x)
  y = None
  sts = []
  for s in range(N_SLICES):
    g = _sc_gather(xp, idx[s])
    y, st = _tc_matmul_stats_slice(x, g, wt, y, s)
    sts.append(st)
  gb = jnp.zeros((8, C_FEAT), jnp.float32).at[0].set(gamma).at[1].set(beta)
  return _tc_bn_relu(y, sts, gb)


# speedup vs baseline: 1.2411x; 1.1030x over previous
import functools

import jax
import jax.numpy as jnp
from jax import lax
from jax.experimental import pallas as pl
from jax.experimental.pallas import tpu as pltpu
from jax.experimental.pallas import tpu_sc as plsc

E_EDGES = 160000
C_FEAT = 128
C_PACK = C_FEAT // 2
NB = 4

NUM_CORES = 2
NUM_SUBCORES = 16
NUM_WORKERS = NUM_CORES * NUM_SUBCORES
DCHUNK = 40
PCHUNK = 40

N_SLICES = 5
E_SLICE = E_EDGES // N_SLICES

EB = 640
GRID = E_EDGES // EB
SBLK = E_SLICE // EB

_SC_PARAMS = pltpu.CompilerParams(use_tc_tiling_on_sc=False)
_MESH = dict(core_axis_name="c", subcore_axis_name="s",
             num_cores=NUM_CORES, num_subcores=NUM_SUBCORES)


def _to_bf16_bits(u):
  one = jnp.full((16,), 1, jnp.uint32)
  rnd = jnp.full((16,), 0x7FFF, jnp.uint32)
  s16 = jnp.full((16,), 16, jnp.uint32)
  return (u + rnd + ((u >> s16) & one)) >> s16


def _sc_pack(x):
  n_rows = x.shape[0]
  per_w = n_rows // NUM_WORKERS
  n_chunks = per_w // PCHUNK
  assert per_w * NUM_WORKERS == n_rows and n_chunks * PCHUNK == per_w
  mesh = plsc.VectorSubcoreMesh(**_MESH)

  @functools.partial(
      pl.kernel,
      mesh=mesh,
      out_type=jax.ShapeDtypeStruct((n_rows, C_PACK), jnp.int32),
      scratch_types=[
          pltpu.VMEM((2, PCHUNK, C_FEAT), jnp.float32),
          pltpu.VMEM((PCHUNK, C_PACK), jnp.int32),
          pltpu.SemaphoreType.DMA,
          pltpu.SemaphoreType.DMA,
      ],
      compiler_params=_SC_PARAMS,
  )
  def k(x_hbm, out_hbm, xin, xout, sem0, sem1):
    wid = lax.axis_index("s") * NUM_CORES + lax.axis_index("c")
    base = pl.multiple_of(wid * per_w, PCHUNK)
    sems = (sem0, sem1)
    s16c = jnp.full((16,), 16, jnp.uint32)

    def start_load(c, b):
      start = pl.multiple_of(base + c * PCHUNK, PCHUNK)
      pltpu.make_async_copy(x_hbm.at[pl.ds(start, PCHUNK)], xin.at[b],
                            sems[b]).start()

    def wait_load(b):
      pltpu.make_async_copy(x_hbm.at[pl.ds(0, PCHUNK)], xin.at[b],
                            sems[b]).wait()

    def pack_chunk(c, b):
      for r in range(PCHUNK):
        for g in range(C_PACK // 16):
          a = xin[b, r, pl.ds(16 * g, 16)]
          cc = xin[b, r, pl.ds(C_PACK + 16 * g, 16)]
          lo = _to_bf16_bits(jax.lax.bitcast_convert_type(a, jnp.uint32))
          hi = _to_bf16_bits(jax.lax.bitcast_convert_type(cc, jnp.uint32))
          xout[r, pl.ds(16 * g, 16)] = jax.lax.bitcast_convert_type(
              (hi << s16c) | lo, jnp.int32)
      start = pl.multiple_of(base + c * PCHUNK, PCHUNK)
      pltpu.sync_copy(xout, out_hbm.at[pl.ds(start, PCHUNK)])

    for b in (0, 1):
      start_load(b, b)

    def body(j, carry):
      for b in (0, 1):
        c = 2 * j + b
        wait_load(b)
        pack_chunk(c, b)
        start_load(c + 2, b)
      return carry

    lax.fori_loop(0, n_chunks // 2 - 1, body, 0)
    for b in (0, 1):
      wait_load(b)
      pack_chunk(n_chunks - 2 + b, b)

  return k(x)


def _sc_gather(xp, idx):
  total = idx.shape[0]
  n_out = total // 2
  per_w = n_out // NUM_WORKERS
  n_chunks = per_w // DCHUNK
  assert per_w * NUM_WORKERS == n_out and n_chunks * DCHUNK == per_w
  mesh = plsc.VectorSubcoreMesh(**_MESH)

  @functools.partial(
      pl.kernel,
      mesh=mesh,
      out_type=jax.ShapeDtypeStruct((n_out, C_FEAT), jnp.int32),
      scratch_types=[
          pltpu.VMEM((2, 2 * DCHUNK), jnp.int32),
          pltpu.VMEM((2, DCHUNK, C_PACK), jnp.int32),
          pltpu.VMEM((2, DCHUNK, C_PACK), jnp.int32),
          pltpu.SemaphoreType.DMA,
          pltpu.SemaphoreType.DMA,
      ],
      compiler_params=_SC_PARAMS,
  )
  def k(x_hbm, idx_hbm, out_hbm, idx_v, rows_l, rows_r, gsem0, gsem1):
    wid = lax.axis_index("s") * NUM_CORES + lax.axis_index("c")
    base = pl.multiple_of(wid * per_w, DCHUNK)
    gsems = (gsem0, gsem1)
    emax = jnp.full((16,), E_EDGES - 1, jnp.int32)
    ezero = jnp.zeros((16,), jnp.int32)

    def halves(b):
      left = idx_v.at[b, pl.ds(0, DCHUNK)]
      right = idx_v.at[b, pl.ds(DCHUNK, DCHUNK)]
      return ((left, rows_l.at[b]), (right, rows_r.at[b]))

    def load_idx_and_start(c, b):
      start = pl.multiple_of(2 * base + c * 2 * DCHUNK, 2 * DCHUNK)
      pltpu.sync_copy(idx_hbm.at[pl.ds(start, 2 * DCHUNK)], idx_v.at[b])
      ib = idx_v.at[b]
      for v in range(2 * DCHUNK // 16):
        seg = ib[pl.ds(v * 16, 16)]
        ib[pl.ds(v * 16, 16)] = jnp.minimum(jnp.maximum(seg, ezero), emax)
      for iv, dv in halves(b):
        pltpu.make_async_copy(x_hbm.at[iv], dv, gsems[b]).start()

    def wait_and_writeback(c, b):
      for iv, dv in halves(b):
        pltpu.make_async_copy(x_hbm.at[iv], dv, gsems[b]).wait()
      start = pl.multiple_of(base + c * DCHUNK, DCHUNK)
      pltpu.sync_copy(rows_l.at[b],
                      out_hbm.at[pl.ds(start, DCHUNK), pl.ds(0, C_PACK)])
      pltpu.sync_copy(rows_r.at[b],
                      out_hbm.at[pl.ds(start, DCHUNK), pl.ds(C_PACK, C_PACK)])

    for b in (0, 1):
      load_idx_and_start(b, b)

    def body(j, carry):
      for b in (0, 1):
        c = 2 * j + b
        wait_and_writeback(c, b)
        load_idx_and_start(c + 2, b)
      return carry

    lax.fori_loop(0, n_chunks // 2 - 1, body, 0)
    for b in (0, 1):
      wait_and_writeback(n_chunks - 2 + b, b)

  return k(xp, idx)


def _split2(d):
  u = jax.lax.bitcast_convert_type(d, jnp.uint32)
  vlo = jax.lax.bitcast_convert_type(
      (u & jnp.uint32(0xFFFF)).astype(jnp.uint16), jnp.bfloat16)
  vhi = jax.lax.bitcast_convert_type(
      (u >> 16).astype(jnp.uint16), jnp.bfloat16)
  a = jnp.concatenate([vlo[:, :C_PACK], vhi[:, :C_PACK]], axis=1)
  b = jnp.concatenate([vlo[:, C_PACK:], vhi[:, C_PACK:]], axis=1)
  return a, b


def _mm_body(x_ref, d1_ref, d2_ref, wt_ref, *rest):
  if len(rest) == 3:
    _, y_ref, st_ref = rest
  else:
    y_ref, st_ref = rest
  i = pl.program_id(0)
  n0, n1 = _split2(d1_ref[...])
  n2, n3 = _split2(d2_ref[...])
  feats = (x_ref[...].astype(jnp.bfloat16),
           jnp.minimum(n0, n1), jnp.maximum(n0, n1),
           jnp.minimum(n2, n3), jnp.maximum(n2, n3))
  y = jnp.zeros((EB, C_FEAT), jnp.float32)
  for j, f in enumerate(feats):
    y = y + jnp.dot(f, wt_ref[j * C_FEAT:(j + 1) * C_FEAT, :],
                    preferred_element_type=jnp.float32)
  y_ref[...] = y.astype(jnp.bfloat16)

  @pl.when(i == 0)
  def _():
    st_ref[...] = jnp.zeros_like(st_ref)

  st_ref[0:1, :] += jnp.sum(y, axis=0, keepdims=True)
  st_ref[1:2, :] += jnp.sum(y * y, axis=0, keepdims=True)


def _tc_matmul_stats_slice(x, g, wt, y_prev, s):
  in_specs = [
      pl.BlockSpec((EB, C_FEAT), lambda i: (s * SBLK + i, 0)),
      pl.BlockSpec((EB, C_FEAT), lambda i: (i, 0)),
      pl.BlockSpec((EB, C_FEAT), lambda i: (SBLK + i, 0)),
      pl.BlockSpec((5 * C_FEAT, C_FEAT), lambda i: (0, 0)),
  ]
  args = [x, g, g, wt]
  kwargs = {}
  if y_prev is not None:
    in_specs.append(pl.BlockSpec((8, C_FEAT), lambda i: (0, 0)))
    args.append(y_prev)
    kwargs["input_output_aliases"] = {4: 0}
  return pl.pallas_call(
      _mm_body,
      grid=(SBLK,),
      in_specs=in_specs,
      out_specs=[
          pl.BlockSpec((EB, C_FEAT), lambda i: (s * SBLK + i, 0)),
          pl.BlockSpec((8, C_FEAT), lambda i: (0, 0)),
      ],
      out_shape=[
          jax.ShapeDtypeStruct((E_EDGES, C_FEAT), jnp.bfloat16),
          jax.ShapeDtypeStruct((8, C_FEAT), jnp.float32),
      ],
      **kwargs,
  )(*args)


def _bn_body(y_ref, st0, st1, st2, st3, st4, gb_ref, o_ref):
  st = st0[...] + st1[...] + st2[...] + st3[...] + st4[...]
  inv_e = jnp.float32(1.0 / E_EDGES)
  mean = st[0, :] * inv_e
  var = st[1, :] * inv_e - mean * mean
  inv = lax.rsqrt(var + 1e-5)
  scale = gb_ref[0, :] * inv
  shift = gb_ref[1, :] - mean * scale
  yv = y_ref[...].astype(jnp.float32)
  o_ref[...] = jnp.maximum(yv * scale[None, :] + shift[None, :], 0.0)


def _tc_bn_relu(y, sts, gb):
  small = pl.BlockSpec((8, C_FEAT), lambda i: (0, 0))
  return pl.pallas_call(
      _bn_body,
      grid=(GRID,),
      in_specs=[pl.BlockSpec((EB, C_FEAT), lambda i: (i, 0))]
      + [small] * (len(sts) + 1),
      out_specs=pl.BlockSpec((EB, C_FEAT), lambda i: (i, 0)),
      out_shape=jax.ShapeDtypeStruct((E_EDGES, C_FEAT), jnp.float32),
  )(y, *sts, gb)


def kernel(x, nb, W, gamma, beta):
  assert E_SLICE % DCHUNK == 0
  nch = E_SLICE // DCHUNK
  nbs = nb.astype(jnp.int32).reshape(N_SLICES, nch, DCHUNK, NB)
  idx = jnp.concatenate(
      [nbs[..., 0:2].transpose(0, 1, 3, 2).reshape(N_SLICES, 2 * E_SLICE),
       nbs[..., 2:4].transpose(0, 1, 3, 2).reshape(N_SLICES, 2 * E_SLICE)],
      axis=1)
  wt = W.T.astype(jnp.bfloat16)
  xp = _sc_pack(x)
  y = None
  sts = []
  for s in range(N_SLICES):
    g = _sc_gather(xp, idx[s])
    y, st = _tc_matmul_stats_slice(x, g, wt, y, s)
    sts.append(st)
  gb = jnp.zeros((8, C_FEAT), jnp.float32).at[0].set(gamma).at[1].set(beta)
  return _tc_bn_relu(y, sts, gb)
